# initial kernel scaffold (unmeasured)
import jax
import jax.numpy as jnp
from jax import lax
from jax.experimental import pallas as pl
from jax.experimental.pallas import tpu as pltpu

N_EXPERTS = 8
E_LOCAL = 4
TOPK = 2


def _peer():
    my_x = lax.axis_index("x")
    my_y = lax.axis_index("y")
    my_z = lax.axis_index("z")
    return my_x, (1 - my_x, my_y, my_z)


def _pair_barrier(peer):
    barrier_sem = pltpu.get_barrier_semaphore()
    pl.semaphore_signal(
        barrier_sem, inc=1, device_id=peer, device_id_type=pl.DeviceIdType.MESH
    )
    pl.semaphore_wait(barrier_sem, 1)


def _exchange_body(x_ref, r_ref, xf_ref, rparts_ref, send_sems, recv_sems):
    my_x, peer = _peer()
    _pair_barrier(peer)

    t = x_ref.shape[0]
    xf_ref[pl.ds(my_x * t, t), :] = x_ref[...]
    rparts_ref[my_x] = r_ref[...]

    rdma_x = pltpu.make_async_remote_copy(
        src_ref=x_ref,
        dst_ref=xf_ref.at[pl.ds(my_x * t, t), :],
        send_sem=send_sems.at[0],
        recv_sem=recv_sems.at[0],
        device_id=peer,
        device_id_type=pl.DeviceIdType.MESH,
    )
    rdma_r = pltpu.make_async_remote_copy(
        src_ref=r_ref,
        dst_ref=rparts_ref.at[my_x],
        send_sem=send_sems.at[1],
        recv_sem=recv_sems.at[1],
        device_id=peer,
        device_id_type=pl.DeviceIdType.MESH,
    )
    rdma_x.start()
    rdma_r.start()
    rdma_x.wait()
    rdma_r.wait()


def _exchange(x, router):
    t, d = x.shape
    return pl.pallas_call(
        _exchange_body,
        out_shape=(
            jax.ShapeDtypeStruct((2 * t, d), x.dtype),
            jax.ShapeDtypeStruct((2,) + router.shape, router.dtype),
        ),
        in_specs=[
            pl.BlockSpec(memory_space=pltpu.VMEM),
            pl.BlockSpec(memory_space=pltpu.VMEM),
        ],
        out_specs=(
            pl.BlockSpec(memory_space=pltpu.VMEM),
            pl.BlockSpec(memory_space=pltpu.VMEM),
        ),
        scratch_shapes=[
            pltpu.SemaphoreType.DMA((2,)),
            pltpu.SemaphoreType.DMA((2,)),
        ],
        compiler_params=pltpu.CompilerParams(collective_id=0),
    )(x, router)


def _routing_weights(xf, rparts, my_x):
    rfull = jnp.concatenate([rparts[0], rparts[1]], axis=1)
    gates = xf @ rfull
    vals, idx = lax.top_k(gates, TOPK)
    w = jax.nn.softmax(vals, axis=-1)
    t = gates.shape[0]
    wfull = jnp.zeros((t, N_EXPERTS), xf.dtype)
    wfull = wfull.at[jnp.arange(t)[:, None], idx].set(w)
    return lax.dynamic_slice(wfull, (0, my_x * E_LOCAL), (t, E_LOCAL))


def _ffn_body(xf_ref, w_ref, w1_ref, w2_ref, out_ref):
    e = pl.program_id(0)
    f = pl.program_id(1)

    @pl.when(jnp.logical_and(e == 0, f == 0))
    def _():
        out_ref[...] = jnp.zeros_like(out_ref)

    onehot = (
        lax.broadcasted_iota(jnp.int32, (1, E_LOCAL), 1) == e
    ).astype(xf_ref.dtype)
    w_col = jnp.sum(w_ref[...] * onehot, axis=1, keepdims=True)

    h = jnp.maximum(
        jnp.dot(xf_ref[...], w1_ref[0], preferred_element_type=jnp.float32), 0.0
    )
    y = jnp.dot(h, w2_ref[0], preferred_element_type=jnp.float32)
    out_ref[...] += w_col * y


def _ffn(xf, wloc, W1, W2):
    t, d = xf.shape
    f = W1.shape[2]
    n_f = 4
    fc = f // n_f
    return pl.pallas_call(
        _ffn_body,
        grid=(E_LOCAL, n_f),
        out_shape=jax.ShapeDtypeStruct((t, d), jnp.float32),
        in_specs=[
            pl.BlockSpec((t, d), lambda e, j: (0, 0)),
            pl.BlockSpec((t, E_LOCAL), lambda e, j: (0, 0)),
            pl.BlockSpec((1, d, fc), lambda e, j: (e, 0, j)),
            pl.BlockSpec((1, fc, d), lambda e, j: (e, j, 0)),
        ],
        out_specs=pl.BlockSpec((t, d), lambda e, j: (0, 0)),
    )(xf, wloc, W1, W2)


def _combine_body(p_ref, out_ref, comm_ref, send_sem, recv_sem):
    my_x, peer = _peer()
    _pair_barrier(peer)

    t = out_ref.shape[0]
    rdma = pltpu.make_async_remote_copy(
        src_ref=p_ref.at[pl.ds((1 - my_x) * t, t), :],
        dst_ref=comm_ref,
        send_sem=send_sem,
        recv_sem=recv_sem,
        device_id=peer,
        device_id_type=pl.DeviceIdType.MESH,
    )
    rdma.start()
    rdma.wait()
    out_ref[...] = p_ref[pl.ds(my_x * t, t), :] + comm_ref[...]


def _combine(partial):
    t2, d = partial.shape
    t = t2 // 2
    return pl.pallas_call(
        _combine_body,
        out_shape=jax.ShapeDtypeStruct((t, d), partial.dtype),
        in_specs=[pl.BlockSpec(memory_space=pltpu.VMEM)],
        out_specs=pl.BlockSpec(memory_space=pltpu.VMEM),
        scratch_shapes=[
            pltpu.VMEM((t, d), partial.dtype),
            pltpu.SemaphoreType.DMA,
            pltpu.SemaphoreType.DMA,
        ],
        compiler_params=pltpu.CompilerParams(collective_id=1),
    )(partial)


def kernel(x, router, W1, W2):
    my_x = lax.axis_index("x")
    xf, rparts = _exchange(x, router)
    wloc = _routing_weights(xf, rparts, my_x)
    partial = _ffn(xf, wloc, W1, W2)
    return _combine(partial)


# baseline (device time: 118959 ns/iter reference)
import jax
import jax.numpy as jnp
from jax import lax
from jax.experimental import pallas as pl
from jax.experimental.pallas import tpu as pltpu

N_EXPERTS = 8
E_LOCAL = 4
TOPK = 2


def _peer():
    my_x = lax.axis_index("x")
    my_y = lax.axis_index("y")
    my_z = lax.axis_index("z")
    return my_x, (1 - my_x, my_y, my_z)


def _pair_barrier(peer):
    barrier_sem = pltpu.get_barrier_semaphore()
    pl.semaphore_signal(
        barrier_sem, inc=1, device_id=peer, device_id_type=pl.DeviceIdType.MESH
    )
    pl.semaphore_wait(barrier_sem, 1)


def _exchange_body(x_ref, r_ref, xf_ref, rparts_ref, send_sems, recv_sems):
    my_x, peer = _peer()
    _pair_barrier(peer)

    t = x_ref.shape[0]
    xf_ref[pl.ds(my_x * t, t), :] = x_ref[...]
    rparts_ref[my_x] = r_ref[...]

    rdma_x = pltpu.make_async_remote_copy(
        src_ref=x_ref,
        dst_ref=xf_ref.at[pl.ds(my_x * t, t), :],
        send_sem=send_sems.at[0],
        recv_sem=recv_sems.at[0],
        device_id=peer,
        device_id_type=pl.DeviceIdType.MESH,
    )
    rdma_r = pltpu.make_async_remote_copy(
        src_ref=r_ref,
        dst_ref=rparts_ref.at[my_x],
        send_sem=send_sems.at[1],
        recv_sem=recv_sems.at[1],
        device_id=peer,
        device_id_type=pl.DeviceIdType.MESH,
    )
    rdma_x.start()
    rdma_r.start()
    rdma_x.wait()
    rdma_r.wait()


def _exchange(x, router):
    t, d = x.shape
    return pl.pallas_call(
        _exchange_body,
        out_shape=(
            jax.ShapeDtypeStruct((2 * t, d), x.dtype),
            jax.ShapeDtypeStruct((2,) + router.shape, router.dtype),
        ),
        in_specs=[
            pl.BlockSpec(memory_space=pltpu.VMEM),
            pl.BlockSpec(memory_space=pltpu.VMEM),
        ],
        out_specs=(
            pl.BlockSpec(memory_space=pltpu.VMEM),
            pl.BlockSpec(memory_space=pltpu.VMEM),
        ),
        scratch_shapes=[
            pltpu.SemaphoreType.DMA((2,)),
            pltpu.SemaphoreType.DMA((2,)),
        ],
        compiler_params=pltpu.CompilerParams(collective_id=0),
    )(x, router)


def _routing_weights(xf, rparts, my_x):
    rfull = jnp.concatenate([rparts[0], rparts[1]], axis=1)
    gates = jnp.dot(xf, rfull, precision=lax.Precision.HIGHEST)
    vals, idx = lax.top_k(gates, TOPK)
    w = jax.nn.softmax(vals, axis=-1)
    t = gates.shape[0]
    wfull = jnp.zeros((t, N_EXPERTS), xf.dtype)
    wfull = wfull.at[jnp.arange(t)[:, None], idx].set(w)
    return lax.dynamic_slice(wfull, (0, my_x * E_LOCAL), (t, E_LOCAL))


def _ffn_body(xf_ref, w_ref, w1_ref, w2_ref, out_ref):
    e = pl.program_id(0)
    f = pl.program_id(1)

    @pl.when(jnp.logical_and(e == 0, f == 0))
    def _():
        out_ref[...] = jnp.zeros_like(out_ref)

    onehot = (
        lax.broadcasted_iota(jnp.int32, (1, E_LOCAL), 1) == e
    ).astype(xf_ref.dtype)
    w_col = jnp.sum(w_ref[...] * onehot, axis=1, keepdims=True)

    h = jnp.maximum(
        jnp.dot(xf_ref[...], w1_ref[0], preferred_element_type=jnp.float32), 0.0
    )
    y = jnp.dot(h, w2_ref[0], preferred_element_type=jnp.float32)
    out_ref[...] += w_col * y


def _ffn(xf, wloc, W1, W2):
    t, d = xf.shape
    f = W1.shape[2]
    n_f = 4
    fc = f // n_f
    return pl.pallas_call(
        _ffn_body,
        grid=(E_LOCAL, n_f),
        out_shape=jax.ShapeDtypeStruct((t, d), jnp.float32),
        in_specs=[
            pl.BlockSpec((t, d), lambda e, j: (0, 0)),
            pl.BlockSpec((t, E_LOCAL), lambda e, j: (0, 0)),
            pl.BlockSpec((1, d, fc), lambda e, j: (e, 0, j)),
            pl.BlockSpec((1, fc, d), lambda e, j: (e, j, 0)),
        ],
        out_specs=pl.BlockSpec((t, d), lambda e, j: (0, 0)),
    )(xf, wloc, W1, W2)


def _combine_body(p_ref, out_ref, comm_ref, send_sem, recv_sem):
    my_x, peer = _peer()
    _pair_barrier(peer)

    t = out_ref.shape[0]
    rdma = pltpu.make_async_remote_copy(
        src_ref=p_ref.at[pl.ds((1 - my_x) * t, t), :],
        dst_ref=comm_ref,
        send_sem=send_sem,
        recv_sem=recv_sem,
        device_id=peer,
        device_id_type=pl.DeviceIdType.MESH,
    )
    rdma.start()
    rdma.wait()
    out_ref[...] = p_ref[pl.ds(my_x * t, t), :] + comm_ref[...]


def _combine(partial):
    t2, d = partial.shape
    t = t2 // 2
    return pl.pallas_call(
        _combine_body,
        out_shape=jax.ShapeDtypeStruct((t, d), partial.dtype),
        in_specs=[pl.BlockSpec(memory_space=pltpu.VMEM)],
        out_specs=pl.BlockSpec(memory_space=pltpu.VMEM),
        scratch_shapes=[
            pltpu.VMEM((t, d), partial.dtype),
            pltpu.SemaphoreType.DMA,
            pltpu.SemaphoreType.DMA,
        ],
        compiler_params=pltpu.CompilerParams(collective_id=1),
    )(partial)


def kernel(x, router, W1, W2):
    my_x = lax.axis_index("x")
    xf, rparts = _exchange(x, router)
    wloc = _routing_weights(xf, rparts, my_x)
    partial = _ffn(xf, wloc, W1, W2)
    return _combine(partial)
